# baseline (speedup 1.0000x reference)
import functools

import jax
import jax.numpy as jnp
from jax import lax
from jax.experimental import pallas as pl
from jax.experimental.pallas import tpu as pltpu
from jax.experimental.pallas import tpu_sc as plsc

_NC = 2
_NS = 16
_CH = 128


def _bcast_lane(vec, lane):
    idx = jnp.full((16, 1), lane, jnp.int32)
    dnums = lax.GatherDimensionNumbers(
        offset_dims=(), collapsed_slice_dims=(0,), start_index_map=(0,))
    return lax.gather(vec, idx, dnums, (1,),
                      mode=lax.GatherScatterMode.PROMISE_IN_BOUNDS)


def _sc_degree(dst_t, w_t, n):
    nw, jpt, ch = dst_t.shape
    zc = 2000
    assert n % zc == 0
    mesh = plsc.VectorSubcoreMesh(core_axis_name="c", subcore_axis_name="s")

    @functools.partial(
        pl.kernel,
        out_type=[jax.ShapeDtypeStruct((n,), jnp.float32)] * _NC,
        mesh=mesh,
        scratch_types=[
            pltpu.VMEM((jpt, ch), jnp.int32),
            pltpu.VMEM((jpt, ch), jnp.float32),
            pltpu.VMEM((zc,), jnp.float32),
            pltpu.VMEM_SHARED((n,), jnp.float32),
            pltpu.SemaphoreType.DMA,
        ],
    )
    def deg_kernel(dst_hbm, w_hbm, out0_hbm, out1_hbm, idx_v, w_v, z_v, deg_sh, sem):
        cid = lax.axis_index("c")
        sid = lax.axis_index("s")
        wid = cid * _NS + sid

        @pl.when(sid == 0)
        def _():
            def zb(i, carry):
                z_v[pl.ds(i * 16, 16)] = jnp.zeros((16,), jnp.float32)
                return carry

            lax.fori_loop(0, zc // 16, zb, 0)
            for k in range(n // zc):
                pltpu.sync_copy(z_v, deg_sh.at[pl.ds(k * zc, zc)])

        plsc.subcore_barrier()
        pltpu.sync_copy(dst_hbm.at[wid], idx_v)
        pltpu.sync_copy(w_hbm.at[wid], w_v)
        group = 8
        for j0 in range(0, jpt, group):
            descs = [
                pltpu.async_copy(w_v.at[j], deg_sh.at[idx_v.at[j]], sem, add=True)
                for j in range(j0, min(j0 + group, jpt))
            ]
            for dsc in descs:
                dsc.wait()
        plsc.subcore_barrier()

        @pl.when((sid == 0) & (cid == 0))
        def _():
            pltpu.sync_copy(deg_sh, out0_hbm)

        @pl.when((sid == 0) & (cid == 1))
        def _():
            pltpu.sync_copy(deg_sh, out1_hbm)

    return deg_kernel(dst_t, w_t)


def _tc_hs(x, w, deg_t):
    n, d = x.shape
    bn = 1000
    assert n % bn == 0

    def body(x_ref, w_ref, deg_ref, o_ref):
        dsum = 1.0 + deg_ref[:, 0:1] + deg_ref[:, 1:2]
        dinv = jnp.where(dsum > 0, lax.rsqrt(jnp.where(dsum > 0, dsum, 1.0)), 0.0)
        h = jnp.dot(x_ref[...], w_ref[...], preferred_element_type=jnp.float32)
        o_ref[...] = h * dinv

    return pl.pallas_call(
        body,
        grid=(n // bn,),
        in_specs=[
            pl.BlockSpec((bn, d), lambda i: (i, 0)),
            pl.BlockSpec((d, d), lambda i: (0, 0)),
            pl.BlockSpec((bn, _NC), lambda i: (i, 0)),
        ],
        out_specs=pl.BlockSpec((bn, d), lambda i: (i, 0)),
        out_shape=jax.ShapeDtypeStruct((n, d), jnp.float32),
    )(x, w, deg_t)


def _sc_scatter(src_t, dst_t, w_t, hs, n):
    nw, jpt, ch = src_t.shape
    d = hs.shape[1]
    nch = -(-n // ch)
    last_rows = n - (nch - 1) * ch
    kmax = -(-nch // _NS)
    sg = 8
    assert jpt % sg == 0
    nsc = jpt // sg
    mesh = plsc.VectorSubcoreMesh(core_axis_name="c", subcore_axis_name="s")

    @functools.partial(
        pl.kernel,
        out_type=[jax.ShapeDtypeStruct((n, d), jnp.float32)] * _NC,
        mesh=mesh,
        scratch_types=[
            pltpu.VMEM((jpt, ch), jnp.int32),
            pltpu.VMEM((jpt, ch), jnp.int32),
            pltpu.VMEM((jpt, ch), jnp.float32),
            pltpu.VMEM((ch, d), jnp.float32),
            pltpu.VMEM_SHARED((n, d), jnp.float32),
            pltpu.SemaphoreType.DMA,
        ],
    )
    def scat_kernel(src_hbm, dst_hbm, w_hbm, hs_hbm, out0_hbm, out1_hbm,
                    src_v, dst_v, w_v, rows_v, s_sh, sem):
        cid = lax.axis_index("c")
        sid = lax.axis_index("s")
        wid = cid * _NS + sid

        def zrow(r, carry):
            for cb in range(d // 16):
                rows_v[r, pl.ds(cb * 16, 16)] = jnp.zeros((16,), jnp.float32)
            return carry

        lax.fori_loop(0, ch, zrow, 0)

        def zchunk(k, carry):
            q = k * _NS + sid

            @pl.when(q < nch - 1)
            def _():
                off = pl.multiple_of(q * ch, 8)
                pltpu.sync_copy(rows_v, s_sh.at[pl.ds(off, ch), :])

            @pl.when(q == nch - 1)
            def _():
                off = pl.multiple_of(q * ch, 8)
                pltpu.sync_copy(rows_v.at[pl.ds(0, last_rows)],
                                s_sh.at[pl.ds(off, last_rows), :])

            return carry

        lax.fori_loop(0, kmax, zchunk, 0)
        plsc.subcore_barrier()

        pltpu.sync_copy(src_hbm.at[wid], src_v)
        pltpu.sync_copy(dst_hbm.at[wid], dst_v)
        pltpu.sync_copy(w_hbm.at[wid], w_v)

        def chunk(j, carry):
            pltpu.async_copy(hs_hbm.at[src_v.at[j]], rows_v, sem).wait()

            def scale(rb, c2):
                wrow = w_v[j, pl.ds(rb * 16, 16)]
                for u in range(16):
                    wv = _bcast_lane(wrow, u)
                    r = rb * 16 + u
                    for cb in range(d // 16):
                        sl = pl.ds(cb * 16, 16)
                        rows_v[r, sl] = rows_v[r, sl] * wv
                return c2

            lax.fori_loop(0, ch // 16, scale, 0)
            pltpu.sync_copy(rows_v, s_sh.at[dst_v.at[j]], add=True)
            return carry

        lax.fori_loop(0, jpt, chunk, 0)
        plsc.subcore_barrier()

        def copy_out(out_hbm):
            def cchunk(k, carry):
                q = k * _NS + sid

                @pl.when(q < nch - 1)
                def _():
                    off = pl.multiple_of(q * ch, 8)
                    pltpu.sync_copy(s_sh.at[pl.ds(off, ch), :],
                                    out_hbm.at[pl.ds(off, ch), :])

                @pl.when(q == nch - 1)
                def _():
                    off = pl.multiple_of(q * ch, 8)
                    pltpu.sync_copy(s_sh.at[pl.ds(off, last_rows), :],
                                    out_hbm.at[pl.ds(off, last_rows), :])

                return carry

            lax.fori_loop(0, kmax, cchunk, 0)

        @pl.when(cid == 0)
        def _():
            copy_out(out0_hbm)

        @pl.when(cid == 1)
        def _():
            copy_out(out1_hbm)

    return scat_kernel(src_t, dst_t, w_t, hs)


def _tc_final(s0, s1, hs, deg_t, batch2, edge_pool, b2, wp, bp2):
    n, d = hs.shape
    g, de = edge_pool.shape
    c = wp.shape[1]
    bn = 1000
    assert n % bn == 0

    def body(s0_ref, s1_ref, hs_ref, deg_ref, bt_ref, ep_ref, b_ref, wp_ref,
             bp_ref, o_ref, sums, counts):
        i = pl.program_id(0)

        @pl.when(i == 0)
        def _():
            sums[...] = jnp.zeros_like(sums)
            counts[...] = jnp.zeros_like(counts)

        dsum = 1.0 + deg_ref[:, 0:1] + deg_ref[:, 1:2]
        dinv = jnp.where(dsum > 0, lax.rsqrt(jnp.where(dsum > 0, dsum, 1.0)), 0.0)
        stot = s0_ref[...] + s1_ref[...]
        structural = jnp.maximum(dinv * (stot + hs_ref[...]) + b_ref[...], 0.0)
        onehot = (bt_ref[...] == lax.broadcasted_iota(jnp.int32, (bn, g), 1))
        onehot = onehot.astype(jnp.float32)
        sums[...] += lax.dot_general(
            onehot, structural, (((0,), (0,)), ((), ())),
            preferred_element_type=jnp.float32)
        counts[...] += lax.dot_general(
            onehot, jnp.ones((bn, 1), jnp.float32), (((0,), (0,)), ((), ())),
            preferred_element_type=jnp.float32)

        @pl.when(i == pl.num_programs(0) - 1)
        def _():
            gr = sums[...] / jnp.maximum(counts[...], 1.0)
            logits = jnp.dot(gr, wp_ref[0:d, :], preferred_element_type=jnp.float32)
            logits += jnp.dot(ep_ref[...], wp_ref[d:d + de, :],
                              preferred_element_type=jnp.float32)
            o_ref[...] = logits + bp_ref[...]

    return pl.pallas_call(
        body,
        grid=(n // bn,),
        in_specs=[
            pl.BlockSpec((bn, d), lambda i: (i, 0)),
            pl.BlockSpec((bn, d), lambda i: (i, 0)),
            pl.BlockSpec((bn, d), lambda i: (i, 0)),
            pl.BlockSpec((bn, _NC), lambda i: (i, 0)),
            pl.BlockSpec((bn, 1), lambda i: (i, 0)),
            pl.BlockSpec((g, de), lambda i: (0, 0)),
            pl.BlockSpec((1, d), lambda i: (0, 0)),
            pl.BlockSpec((d + de, c), lambda i: (0, 0)),
            pl.BlockSpec((1, c), lambda i: (0, 0)),
        ],
        out_specs=pl.BlockSpec((g, c), lambda i: (0, 0)),
        out_shape=jax.ShapeDtypeStruct((g, c), jnp.float32),
        scratch_shapes=[
            pltpu.VMEM((g, d), jnp.float32),
            pltpu.VMEM((g, 1), jnp.float32),
        ],
    )(s0, s1, hs, deg_t, batch2, edge_pool, b2, wp, bp2)


def kernel(x, edge_index, weights, batch, edge_pool, W, b, Wp, bp):
    n, d = x.shape
    e = weights.shape[0]
    nw = _NC * _NS
    jpt = -(-e // (nw * _CH))
    jpt = -(-jpt // 8) * 8
    epad = nw * _CH * jpt
    pad = epad - e

    src = edge_index[0].astype(jnp.int32)
    dst = edge_index[1].astype(jnp.int32)
    src_t = jnp.pad(src, (0, pad)).reshape(nw, jpt, _CH)
    dst_t = jnp.pad(dst, (0, pad)).reshape(nw, jpt, _CH)
    w_t = jnp.pad(weights, (0, pad)).reshape(nw, jpt, _CH)

    deg0, deg1 = _sc_degree(dst_t, w_t, n)
    deg_t = jnp.stack([deg0, deg1], axis=1)
    hs = _tc_hs(x, W, deg_t)
    s0, s1 = _sc_scatter(src_t, dst_t, w_t, hs, n)
    return _tc_final(
        s0, s1, hs, deg_t,
        batch.astype(jnp.int32).reshape(n, 1),
        edge_pool, b.reshape(1, d), Wp, bp.reshape(1, -1),
    )


# speedup vs baseline: 2.4291x; 2.4291x over previous
import functools

import jax
import jax.numpy as jnp
from jax import lax
from jax.experimental import pallas as pl
from jax.experimental.pallas import tpu as pltpu
from jax.experimental.pallas import tpu_sc as plsc

_NC = 2
_NS = 16
_CH = 128


def _bcast_lane(vec, lane):
    idx = jnp.full((16, 1), lane, jnp.int32)
    dnums = lax.GatherDimensionNumbers(
        offset_dims=(), collapsed_slice_dims=(0,), start_index_map=(0,))
    return lax.gather(vec, idx, dnums, (1,),
                      mode=lax.GatherScatterMode.PROMISE_IN_BOUNDS)


def _sc_degree(dst_t, w_t, n):
    nw, jpt, ch = dst_t.shape
    zc = 2000
    assert n % zc == 0
    mesh = plsc.VectorSubcoreMesh(core_axis_name="c", subcore_axis_name="s")

    @functools.partial(
        pl.kernel,
        out_type=[jax.ShapeDtypeStruct((n,), jnp.float32)] * _NC,
        mesh=mesh,
        scratch_types=[
            pltpu.VMEM((jpt, ch), jnp.int32),
            pltpu.VMEM((jpt, ch), jnp.float32),
            pltpu.VMEM((zc,), jnp.float32),
            pltpu.VMEM_SHARED((n,), jnp.float32),
            pltpu.SemaphoreType.DMA,
        ],
    )
    def deg_kernel(dst_hbm, w_hbm, out0_hbm, out1_hbm, idx_v, w_v, z_v, deg_sh, sem):
        cid = lax.axis_index("c")
        sid = lax.axis_index("s")
        wid = cid * _NS + sid

        @pl.when(sid == 0)
        def _():
            def zb(i, carry):
                z_v[pl.ds(i * 16, 16)] = jnp.zeros((16,), jnp.float32)
                return carry

            lax.fori_loop(0, zc // 16, zb, 0)
            for k in range(n // zc):
                pltpu.sync_copy(z_v, deg_sh.at[pl.ds(k * zc, zc)])

        plsc.subcore_barrier()
        pltpu.sync_copy(dst_hbm.at[wid], idx_v)
        pltpu.sync_copy(w_hbm.at[wid], w_v)
        group = 8
        for j0 in range(0, jpt, group):
            descs = [
                pltpu.async_copy(w_v.at[j], deg_sh.at[idx_v.at[j]], sem, add=True)
                for j in range(j0, min(j0 + group, jpt))
            ]
            for dsc in descs:
                dsc.wait()
        plsc.subcore_barrier()

        @pl.when((sid == 0) & (cid == 0))
        def _():
            pltpu.sync_copy(deg_sh, out0_hbm)

        @pl.when((sid == 0) & (cid == 1))
        def _():
            pltpu.sync_copy(deg_sh, out1_hbm)

    return deg_kernel(dst_t, w_t)


def _tc_hs(x, w, deg_t):
    n, d = x.shape
    bn = 1000
    assert n % bn == 0

    def body(x_ref, w_ref, deg_ref, o_ref):
        dsum = 1.0 + deg_ref[:, 0:1] + deg_ref[:, 1:2]
        dinv = jnp.where(dsum > 0, lax.rsqrt(jnp.where(dsum > 0, dsum, 1.0)), 0.0)
        h = jnp.dot(x_ref[...], w_ref[...], preferred_element_type=jnp.float32)
        o_ref[...] = h * dinv

    return pl.pallas_call(
        body,
        grid=(n // bn,),
        in_specs=[
            pl.BlockSpec((bn, d), lambda i: (i, 0)),
            pl.BlockSpec((d, d), lambda i: (0, 0)),
            pl.BlockSpec((bn, _NC), lambda i: (i, 0)),
        ],
        out_specs=pl.BlockSpec((bn, d), lambda i: (i, 0)),
        out_shape=jax.ShapeDtypeStruct((n, d), jnp.float32),
    )(x, w, deg_t)


def _sc_scatter(src_t, dst_t, w_t, hs, n):
    nw, jpt, ch = src_t.shape
    d = hs.shape[1]
    nch = -(-n // ch)
    last_rows = n - (nch - 1) * ch
    kmax = -(-nch // _NS)
    sg = 8
    assert jpt % sg == 0
    nsc = jpt // sg
    mesh = plsc.VectorSubcoreMesh(core_axis_name="c", subcore_axis_name="s")

    @functools.partial(
        pl.kernel,
        out_type=[jax.ShapeDtypeStruct((n, d), jnp.float32)] * _NC,
        mesh=mesh,
        scratch_types=[
            pltpu.VMEM((jpt, ch), jnp.int32),
            pltpu.VMEM((jpt, ch), jnp.int32),
            pltpu.VMEM((jpt, ch), jnp.float32),
            pltpu.VMEM((ch, d), jnp.float32),
            pltpu.VMEM_SHARED((n, d), jnp.float32),
            pltpu.SemaphoreType.DMA,
        ],
    )
    def scat_kernel(src_hbm, dst_hbm, w_hbm, hs_hbm, out0_hbm, out1_hbm,
                    src_v, dst_v, w_v, rows_v, s_sh, sem):
        cid = lax.axis_index("c")
        sid = lax.axis_index("s")
        wid = cid * _NS + sid

        def zrow(r, carry):
            for cb in range(d // 16):
                rows_v[r, pl.ds(cb * 16, 16)] = jnp.zeros((16,), jnp.float32)
            return carry

        lax.fori_loop(0, ch, zrow, 0)

        def zchunk(k, carry):
            q = k * _NS + sid

            @pl.when(q < nch - 1)
            def _():
                off = pl.multiple_of(q * ch, 8)
                pltpu.sync_copy(rows_v, s_sh.at[pl.ds(off, ch), :])

            @pl.when(q == nch - 1)
            def _():
                off = pl.multiple_of(q * ch, 8)
                pltpu.sync_copy(rows_v.at[pl.ds(0, last_rows)],
                                s_sh.at[pl.ds(off, last_rows), :])

            return carry

        lax.fori_loop(0, kmax, zchunk, 0)
        plsc.subcore_barrier()

        pltpu.sync_copy(src_hbm.at[wid], src_v)
        pltpu.sync_copy(dst_hbm.at[wid], dst_v)
        pltpu.sync_copy(w_hbm.at[wid], w_v)

        def chunk(j, carry):
            pltpu.async_copy(hs_hbm.at[src_v.at[j]], rows_v, sem).wait()

            def scale(rb, c2):
                wrow = w_v[j, pl.ds(rb * 16, 16)]
                for u in range(16):
                    wv = _bcast_lane(wrow, u)
                    r = rb * 16 + u
                    for cb in range(d // 16):
                        sl = pl.ds(cb * 16, 16)
                        rows_v[r, sl] = rows_v[r, sl] * wv
                return c2

            lax.fori_loop(0, ch // 16, scale, 0)
            pltpu.sync_copy(rows_v, s_sh.at[dst_v.at[j]], add=True)
            return carry

        lax.fori_loop(0, jpt, chunk, 0)
        plsc.subcore_barrier()

        def copy_out(out_hbm):
            def cchunk(k, carry):
                q = k * _NS + sid

                @pl.when(q < nch - 1)
                def _():
                    off = pl.multiple_of(q * ch, 8)
                    pltpu.sync_copy(s_sh.at[pl.ds(off, ch), :],
                                    out_hbm.at[pl.ds(off, ch), :])

                @pl.when(q == nch - 1)
                def _():
                    off = pl.multiple_of(q * ch, 8)
                    pltpu.sync_copy(s_sh.at[pl.ds(off, last_rows), :],
                                    out_hbm.at[pl.ds(off, last_rows), :])

                return carry

            lax.fori_loop(0, kmax, cchunk, 0)

        @pl.when(cid == 0)
        def _():
            copy_out(out0_hbm)

        @pl.when(cid == 1)
        def _():
            copy_out(out1_hbm)

    return scat_kernel(src_t, dst_t, w_t, hs)


def _tc_final(s0, s1, hs, deg_t, batch2, edge_pool, b2, wp, bp2):
    n, d = hs.shape
    g, de = edge_pool.shape
    c = wp.shape[1]
    bn = 1000
    assert n % bn == 0

    def body(s0_ref, s1_ref, hs_ref, deg_ref, bt_ref, ep_ref, b_ref, wp_ref,
             bp_ref, o_ref, sums, counts):
        i = pl.program_id(0)

        @pl.when(i == 0)
        def _():
            sums[...] = jnp.zeros_like(sums)
            counts[...] = jnp.zeros_like(counts)

        dsum = 1.0 + deg_ref[:, 0:1] + deg_ref[:, 1:2]
        dinv = jnp.where(dsum > 0, lax.rsqrt(jnp.where(dsum > 0, dsum, 1.0)), 0.0)
        stot = s0_ref[...] + s1_ref[...]
        structural = jnp.maximum(dinv * (stot + hs_ref[...]) + b_ref[...], 0.0)
        onehot = (bt_ref[...] == lax.broadcasted_iota(jnp.int32, (bn, g), 1))
        onehot = onehot.astype(jnp.float32)
        sums[...] += lax.dot_general(
            onehot, structural, (((0,), (0,)), ((), ())),
            preferred_element_type=jnp.float32)
        counts[...] += lax.dot_general(
            onehot, jnp.ones((bn, 1), jnp.float32), (((0,), (0,)), ((), ())),
            preferred_element_type=jnp.float32)

        @pl.when(i == pl.num_programs(0) - 1)
        def _():
            gr = sums[...] / jnp.maximum(counts[...], 1.0)
            logits = jnp.dot(gr, wp_ref[0:d, :], preferred_element_type=jnp.float32)
            logits += jnp.dot(ep_ref[...], wp_ref[d:d + de, :],
                              preferred_element_type=jnp.float32)
            o_ref[...] = logits + bp_ref[...]

    return pl.pallas_call(
        body,
        grid=(n // bn,),
        in_specs=[
            pl.BlockSpec((bn, d), lambda i: (i, 0)),
            pl.BlockSpec((bn, d), lambda i: (i, 0)),
            pl.BlockSpec((bn, d), lambda i: (i, 0)),
            pl.BlockSpec((bn, _NC), lambda i: (i, 0)),
            pl.BlockSpec((bn, 1), lambda i: (i, 0)),
            pl.BlockSpec((g, de), lambda i: (0, 0)),
            pl.BlockSpec((1, d), lambda i: (0, 0)),
            pl.BlockSpec((d + de, c), lambda i: (0, 0)),
            pl.BlockSpec((1, c), lambda i: (0, 0)),
        ],
        out_specs=pl.BlockSpec((g, c), lambda i: (0, 0)),
        out_shape=jax.ShapeDtypeStruct((g, c), jnp.float32),
        scratch_shapes=[
            pltpu.VMEM((g, d), jnp.float32),
            pltpu.VMEM((g, 1), jnp.float32),
        ],
    )(s0, s1, hs, deg_t, batch2, edge_pool, b2, wp, bp2)


def kernel(x, edge_index, weights, batch, edge_pool, W, b, Wp, bp):
    n, d = x.shape
    e = weights.shape[0]
    nw = _NC * _NS
    jpt = -(-e // (nw * _CH))
    jpt = -(-jpt // 8) * 8
    epad = nw * _CH * jpt
    pad = epad - e

    src = edge_index[0].astype(jnp.int32)
    dst = edge_index[1].astype(jnp.int32)
    pad_ids = jnp.arange(pad, dtype=jnp.int32) % n
    src_t = jnp.concatenate([src, pad_ids]).reshape(nw, jpt, _CH)
    dst_t = jnp.concatenate([dst, pad_ids]).reshape(nw, jpt, _CH)
    w_t = jnp.pad(weights, (0, pad)).reshape(nw, jpt, _CH)

    deg0, deg1 = _sc_degree(dst_t, w_t, n)
    deg_t = jnp.stack([deg0, deg1], axis=1)
    hs = _tc_hs(x, W, deg_t)
    s0, s1 = _sc_scatter(src_t, dst_t, w_t, hs, n)
    return _tc_final(
        s0, s1, hs, deg_t,
        batch.astype(jnp.int32).reshape(n, 1),
        edge_pool, b.reshape(1, d), Wp, bp.reshape(1, -1),
    )


# speedup vs baseline: 3.5059x; 1.4433x over previous
import functools

import jax
import jax.numpy as jnp
from jax import lax
from jax.experimental import pallas as pl
from jax.experimental.pallas import tpu as pltpu
from jax.experimental.pallas import tpu_sc as plsc

_NC = 2
_NS = 16
_CH = 128


def _bcast_lane(vec, lane):
    idx = jnp.full((16, 1), lane, jnp.int32)
    dnums = lax.GatherDimensionNumbers(
        offset_dims=(), collapsed_slice_dims=(0,), start_index_map=(0,))
    return lax.gather(vec, idx, dnums, (1,),
                      mode=lax.GatherScatterMode.PROMISE_IN_BOUNDS)


def _sc_degree(dst_t, w_t, n):
    nw, jpt, ch = dst_t.shape
    zc = 2000
    assert n % zc == 0
    mesh = plsc.VectorSubcoreMesh(core_axis_name="c", subcore_axis_name="s")

    @functools.partial(
        pl.kernel,
        out_type=[jax.ShapeDtypeStruct((n,), jnp.float32)] * _NC,
        mesh=mesh,
        scratch_types=[
            pltpu.VMEM((jpt, ch), jnp.int32),
            pltpu.VMEM((jpt, ch), jnp.float32),
            pltpu.VMEM((zc,), jnp.float32),
            pltpu.VMEM_SHARED((n,), jnp.float32),
            pltpu.SemaphoreType.DMA,
        ],
    )
    def deg_kernel(dst_hbm, w_hbm, out0_hbm, out1_hbm, idx_v, w_v, z_v, deg_sh, sem):
        cid = lax.axis_index("c")
        sid = lax.axis_index("s")
        wid = cid * _NS + sid

        @pl.when(sid == 0)
        def _():
            def zb(i, carry):
                z_v[pl.ds(i * 16, 16)] = jnp.zeros((16,), jnp.float32)
                return carry

            lax.fori_loop(0, zc // 16, zb, 0)
            for k in range(n // zc):
                pltpu.sync_copy(z_v, deg_sh.at[pl.ds(k * zc, zc)])

        plsc.subcore_barrier()
        pltpu.sync_copy(dst_hbm.at[wid], idx_v)
        pltpu.sync_copy(w_hbm.at[wid], w_v)
        group = 8
        for j0 in range(0, jpt, group):
            descs = [
                pltpu.async_copy(w_v.at[j], deg_sh.at[idx_v.at[j]], sem, add=True)
                for j in range(j0, min(j0 + group, jpt))
            ]
            for dsc in descs:
                dsc.wait()
        plsc.subcore_barrier()

        @pl.when((sid == 0) & (cid == 0))
        def _():
            pltpu.sync_copy(deg_sh, out0_hbm)

        @pl.when((sid == 0) & (cid == 1))
        def _():
            pltpu.sync_copy(deg_sh, out1_hbm)

    return deg_kernel(dst_t, w_t)


def _tc_hs(x, w, deg_t):
    n, d = x.shape
    bn = 1000
    assert n % bn == 0

    def body(x_ref, w_ref, deg_ref, o_ref):
        dsum = 1.0 + deg_ref[:, 0:1] + deg_ref[:, 1:2]
        dinv = jnp.where(dsum > 0, lax.rsqrt(jnp.where(dsum > 0, dsum, 1.0)), 0.0)
        h = jnp.dot(x_ref[...], w_ref[...], preferred_element_type=jnp.float32)
        o_ref[...] = h * dinv

    return pl.pallas_call(
        body,
        grid=(n // bn,),
        in_specs=[
            pl.BlockSpec((bn, d), lambda i: (i, 0)),
            pl.BlockSpec((d, d), lambda i: (0, 0)),
            pl.BlockSpec((bn, _NC), lambda i: (i, 0)),
        ],
        out_specs=pl.BlockSpec((bn, d), lambda i: (i, 0)),
        out_shape=jax.ShapeDtypeStruct((n, d), jnp.float32),
    )(x, w, deg_t)


def _sc_scatter(src_t, dst_t, w_t, hs, n):
    nw, jpt, ch = src_t.shape
    d = hs.shape[1]
    nch = -(-n // ch)
    last_rows = n - (nch - 1) * ch
    kmax = -(-nch // _NS)
    sg = 8
    assert jpt % sg == 0
    nsc = jpt // sg
    mesh = plsc.VectorSubcoreMesh(core_axis_name="c", subcore_axis_name="s")

    @functools.partial(
        pl.kernel,
        out_type=[jax.ShapeDtypeStruct((n, d), jnp.float32)] * _NC,
        mesh=mesh,
        scratch_types=[
            pltpu.VMEM((jpt, ch), jnp.int32),
            pltpu.VMEM((2, sg, ch), jnp.int32),
            pltpu.VMEM((2, sg, ch), jnp.float32),
            pltpu.VMEM((ch, d), jnp.float32),
            pltpu.VMEM((ch, d), jnp.float32),
            pltpu.VMEM_SHARED((n, d), jnp.float32),
            pltpu.SemaphoreType.DMA,
            pltpu.SemaphoreType.DMA,
            pltpu.SemaphoreType.DMA,
        ],
    )
    def scat_kernel(src_hbm, dst_hbm, w_hbm, hs_hbm, out0_hbm, out1_hbm,
                    src_v, dst_v, w_v, rows_a, rows_b, s_sh, sem_a, sem_b,
                    sem_i):
        cid = lax.axis_index("c")
        sid = lax.axis_index("s")
        wid = cid * _NS + sid

        def zrow(r, carry):
            for cb in range(d // 16):
                rows_a[r, pl.ds(cb * 16, 16)] = jnp.zeros((16,), jnp.float32)
            return carry

        lax.fori_loop(0, ch, zrow, 0)

        def zchunk(k, carry):
            q = k * _NS + sid

            @pl.when(q < nch - 1)
            def _():
                off = pl.multiple_of(q * ch, 8)
                pltpu.sync_copy(rows_a, s_sh.at[pl.ds(off, ch), :])

            @pl.when(q == nch - 1)
            def _():
                off = pl.multiple_of(q * ch, 8)
                pltpu.sync_copy(rows_a.at[pl.ds(0, last_rows)],
                                s_sh.at[pl.ds(off, last_rows), :])

            return carry

        lax.fori_loop(0, kmax, zchunk, 0)
        plsc.subcore_barrier()

        pltpu.sync_copy(src_hbm.at[wid], src_v)

        def stage_copies(s, b):
            off = pl.multiple_of(s * sg, 8)
            return [
                (dst_hbm.at[wid, pl.ds(off, sg)], dst_v.at[b]),
                (w_hbm.at[wid, pl.ds(off, sg)], w_v.at[b]),
            ]

        def scale(rows_v, b, k):
            def sbody(rb, c2):
                wrow = w_v[b, k, pl.ds(rb * 16, 16)]
                for u in range(16):
                    wv = _bcast_lane(wrow, u)
                    r = rb * 16 + u
                    for cb in range(d // 16):
                        sl = pl.ds(cb * 16, 16)
                        rows_v[r, sl] = rows_v[r, sl] * wv
                return c2

            lax.fori_loop(0, ch // 16, sbody, 0)

        for s0c, d0c in stage_copies(0, 0):
            pltpu.sync_copy(s0c, d0c)

        pltpu.async_copy(hs_hbm.at[src_v.at[0]], rows_a, sem_a)

        def superchunk(s, carry):
            b = s % 2

            @pl.when(s + 1 < nsc)
            def _():
                for s1c, d1c in stage_copies(s + 1, 1 - b):
                    pltpu.async_copy(s1c, d1c, sem_i)

            def pair(q, c2):
                ja = s * sg + 2 * q
                jb = ja + 1
                pltpu.make_async_copy(hs_hbm.at[src_v.at[ja]], rows_a,
                                      sem_a).wait()
                gb = pltpu.async_copy(hs_hbm.at[src_v.at[jb]], rows_b, sem_b)
                scale(rows_a, b, 2 * q)
                pltpu.sync_copy(rows_a, s_sh.at[dst_v.at[b, 2 * q]], add=True)
                gb.wait()

                @pl.when(ja + 2 < jpt)
                def _():
                    pltpu.async_copy(hs_hbm.at[src_v.at[ja + 2]], rows_a,
                                     sem_a)

                scale(rows_b, b, 2 * q + 1)
                pltpu.sync_copy(rows_b, s_sh.at[dst_v.at[b, 2 * q + 1]],
                                add=True)
                return c2

            lax.fori_loop(0, sg // 2, pair, 0)

            @pl.when(s + 1 < nsc)
            def _():
                for s1c, d1c in stage_copies(s + 1, 1 - b):
                    pltpu.make_async_copy(s1c, d1c, sem_i).wait()

            return carry

        lax.fori_loop(0, nsc, superchunk, 0)
        plsc.subcore_barrier()

        def copy_out(out_hbm):
            def cchunk(k, carry):
                q = k * _NS + sid

                @pl.when(q < nch - 1)
                def _():
                    off = pl.multiple_of(q * ch, 8)
                    pltpu.sync_copy(s_sh.at[pl.ds(off, ch), :],
                                    out_hbm.at[pl.ds(off, ch), :])

                @pl.when(q == nch - 1)
                def _():
                    off = pl.multiple_of(q * ch, 8)
                    pltpu.sync_copy(s_sh.at[pl.ds(off, last_rows), :],
                                    out_hbm.at[pl.ds(off, last_rows), :])

                return carry

            lax.fori_loop(0, kmax, cchunk, 0)

        @pl.when(cid == 0)
        def _():
            copy_out(out0_hbm)

        @pl.when(cid == 1)
        def _():
            copy_out(out1_hbm)

    return scat_kernel(src_t, dst_t, w_t, hs)


def _tc_final(s0, s1, hs, deg_t, batch2, edge_pool, b2, wp, bp2):
    n, d = hs.shape
    g, de = edge_pool.shape
    c = wp.shape[1]
    bn = 1000
    assert n % bn == 0

    def body(s0_ref, s1_ref, hs_ref, deg_ref, bt_ref, ep_ref, b_ref, wp_ref,
             bp_ref, o_ref, sums, counts):
        i = pl.program_id(0)

        @pl.when(i == 0)
        def _():
            sums[...] = jnp.zeros_like(sums)
            counts[...] = jnp.zeros_like(counts)

        dsum = 1.0 + deg_ref[:, 0:1] + deg_ref[:, 1:2]
        dinv = jnp.where(dsum > 0, lax.rsqrt(jnp.where(dsum > 0, dsum, 1.0)), 0.0)
        stot = s0_ref[...] + s1_ref[...]
        structural = jnp.maximum(dinv * (stot + hs_ref[...]) + b_ref[...], 0.0)
        onehot = (bt_ref[...] == lax.broadcasted_iota(jnp.int32, (bn, g), 1))
        onehot = onehot.astype(jnp.float32)
        sums[...] += lax.dot_general(
            onehot, structural, (((0,), (0,)), ((), ())),
            preferred_element_type=jnp.float32)
        counts[...] += lax.dot_general(
            onehot, jnp.ones((bn, 1), jnp.float32), (((0,), (0,)), ((), ())),
            preferred_element_type=jnp.float32)

        @pl.when(i == pl.num_programs(0) - 1)
        def _():
            gr = sums[...] / jnp.maximum(counts[...], 1.0)
            logits = jnp.dot(gr, wp_ref[0:d, :], preferred_element_type=jnp.float32)
            logits += jnp.dot(ep_ref[...], wp_ref[d:d + de, :],
                              preferred_element_type=jnp.float32)
            o_ref[...] = logits + bp_ref[...]

    return pl.pallas_call(
        body,
        grid=(n // bn,),
        in_specs=[
            pl.BlockSpec((bn, d), lambda i: (i, 0)),
            pl.BlockSpec((bn, d), lambda i: (i, 0)),
            pl.BlockSpec((bn, d), lambda i: (i, 0)),
            pl.BlockSpec((bn, _NC), lambda i: (i, 0)),
            pl.BlockSpec((bn, 1), lambda i: (i, 0)),
            pl.BlockSpec((g, de), lambda i: (0, 0)),
            pl.BlockSpec((1, d), lambda i: (0, 0)),
            pl.BlockSpec((d + de, c), lambda i: (0, 0)),
            pl.BlockSpec((1, c), lambda i: (0, 0)),
        ],
        out_specs=pl.BlockSpec((g, c), lambda i: (0, 0)),
        out_shape=jax.ShapeDtypeStruct((g, c), jnp.float32),
        scratch_shapes=[
            pltpu.VMEM((g, d), jnp.float32),
            pltpu.VMEM((g, 1), jnp.float32),
        ],
    )(s0, s1, hs, deg_t, batch2, edge_pool, b2, wp, bp2)


def kernel(x, edge_index, weights, batch, edge_pool, W, b, Wp, bp):
    n, d = x.shape
    e = weights.shape[0]
    nw = _NC * _NS
    jpt = -(-e // (nw * _CH))
    jpt = -(-jpt // 8) * 8
    epad = nw * _CH * jpt
    pad = epad - e

    src = edge_index[0].astype(jnp.int32)
    dst = edge_index[1].astype(jnp.int32)
    pad_ids = jnp.arange(pad, dtype=jnp.int32) % n
    src_t = jnp.concatenate([src, pad_ids]).reshape(nw, jpt, _CH)
    dst_t = jnp.concatenate([dst, pad_ids]).reshape(nw, jpt, _CH)
    w_t = jnp.pad(weights, (0, pad)).reshape(nw, jpt, _CH)

    deg0, deg1 = _sc_degree(dst_t, w_t, n)
    deg_t = jnp.stack([deg0, deg1], axis=1)
    hs = _tc_hs(x, W, deg_t)
    s0, s1 = _sc_scatter(src_t, dst_t, w_t, hs, n)
    return _tc_final(
        s0, s1, hs, deg_t,
        batch.astype(jnp.int32).reshape(n, 1),
        edge_pool, b.reshape(1, d), Wp, bp.reshape(1, -1),
    )


# speedup vs baseline: 3.5475x; 1.0119x over previous
import functools

import jax
import jax.numpy as jnp
from jax import lax
from jax.experimental import pallas as pl
from jax.experimental.pallas import tpu as pltpu
from jax.experimental.pallas import tpu_sc as plsc

_NC = 2
_NS = 16
_CH = 128


def _bcast_lane(vec, lane):
    idx = jnp.full((16, 1), lane, jnp.int32)
    dnums = lax.GatherDimensionNumbers(
        offset_dims=(), collapsed_slice_dims=(0,), start_index_map=(0,))
    return lax.gather(vec, idx, dnums, (1,),
                      mode=lax.GatherScatterMode.PROMISE_IN_BOUNDS)


def _sc_degree(dst_t, w_t, n):
    nw, jpt, ch = dst_t.shape
    zc = 2000
    assert n % zc == 0
    mesh = plsc.VectorSubcoreMesh(core_axis_name="c", subcore_axis_name="s")

    @functools.partial(
        pl.kernel,
        out_type=[jax.ShapeDtypeStruct((n,), jnp.float32)] * _NC,
        mesh=mesh,
        scratch_types=[
            pltpu.VMEM((jpt, ch), jnp.int32),
            pltpu.VMEM((jpt, ch), jnp.float32),
            pltpu.VMEM((zc,), jnp.float32),
            pltpu.VMEM_SHARED((n,), jnp.float32),
            pltpu.SemaphoreType.DMA,
        ],
    )
    def deg_kernel(dst_hbm, w_hbm, out0_hbm, out1_hbm, idx_v, w_v, z_v, deg_sh, sem):
        cid = lax.axis_index("c")
        sid = lax.axis_index("s")
        wid = cid * _NS + sid

        @pl.when(sid == 0)
        def _():
            def zb(i, carry):
                z_v[pl.ds(i * 16, 16)] = jnp.zeros((16,), jnp.float32)
                return carry

            lax.fori_loop(0, zc // 16, zb, 0)
            for k in range(n // zc):
                pltpu.sync_copy(z_v, deg_sh.at[pl.ds(k * zc, zc)])

        plsc.subcore_barrier()
        pltpu.sync_copy(dst_hbm.at[wid], idx_v)
        pltpu.sync_copy(w_hbm.at[wid], w_v)
        group = 8
        for j0 in range(0, jpt, group):
            descs = [
                pltpu.async_copy(w_v.at[j], deg_sh.at[idx_v.at[j]], sem, add=True)
                for j in range(j0, min(j0 + group, jpt))
            ]
            for dsc in descs:
                dsc.wait()
        plsc.subcore_barrier()

        @pl.when((sid == 0) & (cid == 0))
        def _():
            pltpu.sync_copy(deg_sh, out0_hbm)

        @pl.when((sid == 0) & (cid == 1))
        def _():
            pltpu.sync_copy(deg_sh, out1_hbm)

    return deg_kernel(dst_t, w_t)


def _tc_hs(x, w, deg_t):
    n, d = x.shape
    bn = 1000
    assert n % bn == 0

    def body(x_ref, w_ref, deg_ref, o_ref):
        dsum = 1.0 + deg_ref[:, 0:1] + deg_ref[:, 1:2]
        dinv = jnp.where(dsum > 0, lax.rsqrt(jnp.where(dsum > 0, dsum, 1.0)), 0.0)
        h = jnp.dot(x_ref[...], w_ref[...], preferred_element_type=jnp.float32)
        o_ref[...] = h * dinv

    return pl.pallas_call(
        body,
        grid=(n // bn,),
        in_specs=[
            pl.BlockSpec((bn, d), lambda i: (i, 0)),
            pl.BlockSpec((d, d), lambda i: (0, 0)),
            pl.BlockSpec((bn, _NC), lambda i: (i, 0)),
        ],
        out_specs=pl.BlockSpec((bn, d), lambda i: (i, 0)),
        out_shape=jax.ShapeDtypeStruct((n, d), jnp.float32),
    )(x, w, deg_t)


def _sc_scatter(src_t, dst_t, w_t, hs, n):
    nw, jpt, ch = src_t.shape
    d = hs.shape[1]
    nch = -(-n // ch)
    last_rows = n - (nch - 1) * ch
    kmax = -(-nch // _NS)
    sg = 8
    assert jpt % sg == 0
    nsc = jpt // sg
    mesh = plsc.VectorSubcoreMesh(core_axis_name="c", subcore_axis_name="s")

    @functools.partial(
        pl.kernel,
        out_type=[jax.ShapeDtypeStruct((n, d), jnp.float32)] * _NC,
        mesh=mesh,
        scratch_types=[
            pltpu.VMEM((jpt, ch), jnp.int32),
            pltpu.VMEM((2, sg, ch), jnp.int32),
            pltpu.VMEM((2, sg, ch), jnp.float32),
            pltpu.VMEM((ch, d), jnp.float32),
            pltpu.VMEM((ch, d), jnp.float32),
            pltpu.VMEM_SHARED((n, d), jnp.float32),
            pltpu.SemaphoreType.DMA,
            pltpu.SemaphoreType.DMA,
            pltpu.SemaphoreType.DMA,
        ],
    )
    def scat_kernel(src_hbm, dst_hbm, w_hbm, hs_hbm, out0_hbm, out1_hbm,
                    src_v, dst_v, w_v, rows_a, rows_b, s_sh, sem_a, sem_b,
                    sem_i):
        cid = lax.axis_index("c")
        sid = lax.axis_index("s")
        wid = cid * _NS + sid

        def zrow(r, carry):
            for cb in range(d // 16):
                rows_a[r, pl.ds(cb * 16, 16)] = jnp.zeros((16,), jnp.float32)
            return carry

        lax.fori_loop(0, ch, zrow, 0)

        def zchunk(k, carry):
            q = k * _NS + sid

            @pl.when(q < nch - 1)
            def _():
                off = pl.multiple_of(q * ch, 8)
                pltpu.sync_copy(rows_a, s_sh.at[pl.ds(off, ch), :])

            @pl.when(q == nch - 1)
            def _():
                off = pl.multiple_of(q * ch, 8)
                pltpu.sync_copy(rows_a.at[pl.ds(0, last_rows)],
                                s_sh.at[pl.ds(off, last_rows), :])

            return carry

        lax.fori_loop(0, kmax, zchunk, 0)
        plsc.subcore_barrier()

        pltpu.sync_copy(src_hbm.at[wid], src_v)

        def stage_copies(s, b):
            off = pl.multiple_of(s * sg, 8)
            return [
                (dst_hbm.at[wid, pl.ds(off, sg)], dst_v.at[b]),
                (w_hbm.at[wid, pl.ds(off, sg)], w_v.at[b]),
            ]

        def scale(rows_v, b, k):
            def sbody(rb, c2):
                wrow = w_v[b, k, pl.ds(rb * 16, 16)]
                for u in range(16):
                    wv = _bcast_lane(wrow, u)
                    r = rb * 16 + u
                    for cb in range(d // 16):
                        sl = pl.ds(cb * 16, 16)
                        rows_v[r, sl] = rows_v[r, sl] * wv
                return c2

            lax.fori_loop(0, ch // 16, sbody, 0)

        for s0c, d0c in stage_copies(0, 0):
            pltpu.sync_copy(s0c, d0c)

        pltpu.async_copy(hs_hbm.at[src_v.at[0]], rows_a, sem_a)

        def superchunk(s, carry):
            b = s % 2

            @pl.when(s + 1 < nsc)
            def _():
                for s1c, d1c in stage_copies(s + 1, 1 - b):
                    pltpu.async_copy(s1c, d1c, sem_i)

            def pair(q, c2):
                ja = s * sg + 2 * q
                jb = ja + 1
                gb = pltpu.async_copy(hs_hbm.at[src_v.at[jb]], rows_b, sem_b)
                pltpu.make_async_copy(hs_hbm.at[src_v.at[ja]], rows_a,
                                      sem_a).wait()
                scale(rows_a, b, 2 * q)
                pltpu.sync_copy(rows_a, s_sh.at[dst_v.at[b, 2 * q]], add=True)

                @pl.when(ja + 2 < jpt)
                def _():
                    pltpu.async_copy(hs_hbm.at[src_v.at[ja + 2]], rows_a,
                                     sem_a)

                gb.wait()
                scale(rows_b, b, 2 * q + 1)
                pltpu.sync_copy(rows_b, s_sh.at[dst_v.at[b, 2 * q + 1]],
                                add=True)
                return c2

            lax.fori_loop(0, sg // 2, pair, 0)

            @pl.when(s + 1 < nsc)
            def _():
                for s1c, d1c in stage_copies(s + 1, 1 - b):
                    pltpu.make_async_copy(s1c, d1c, sem_i).wait()

            return carry

        lax.fori_loop(0, nsc, superchunk, 0)
        plsc.subcore_barrier()

        def copy_out(out_hbm):
            def cchunk(k, carry):
                q = k * _NS + sid

                @pl.when(q < nch - 1)
                def _():
                    off = pl.multiple_of(q * ch, 8)
                    pltpu.sync_copy(s_sh.at[pl.ds(off, ch), :],
                                    out_hbm.at[pl.ds(off, ch), :])

                @pl.when(q == nch - 1)
                def _():
                    off = pl.multiple_of(q * ch, 8)
                    pltpu.sync_copy(s_sh.at[pl.ds(off, last_rows), :],
                                    out_hbm.at[pl.ds(off, last_rows), :])

                return carry

            lax.fori_loop(0, kmax, cchunk, 0)

        @pl.when(cid == 0)
        def _():
            copy_out(out0_hbm)

        @pl.when(cid == 1)
        def _():
            copy_out(out1_hbm)

    return scat_kernel(src_t, dst_t, w_t, hs)


def _tc_final(s0, s1, hs, deg_t, batch2, edge_pool, b2, wp, bp2):
    n, d = hs.shape
    g, de = edge_pool.shape
    c = wp.shape[1]
    bn = 1000
    assert n % bn == 0

    def body(s0_ref, s1_ref, hs_ref, deg_ref, bt_ref, ep_ref, b_ref, wp_ref,
             bp_ref, o_ref, sums, counts):
        i = pl.program_id(0)

        @pl.when(i == 0)
        def _():
            sums[...] = jnp.zeros_like(sums)
            counts[...] = jnp.zeros_like(counts)

        dsum = 1.0 + deg_ref[:, 0:1] + deg_ref[:, 1:2]
        dinv = jnp.where(dsum > 0, lax.rsqrt(jnp.where(dsum > 0, dsum, 1.0)), 0.0)
        stot = s0_ref[...] + s1_ref[...]
        structural = jnp.maximum(dinv * (stot + hs_ref[...]) + b_ref[...], 0.0)
        onehot = (bt_ref[...] == lax.broadcasted_iota(jnp.int32, (bn, g), 1))
        onehot = onehot.astype(jnp.float32)
        sums[...] += lax.dot_general(
            onehot, structural, (((0,), (0,)), ((), ())),
            preferred_element_type=jnp.float32)
        counts[...] += lax.dot_general(
            onehot, jnp.ones((bn, 1), jnp.float32), (((0,), (0,)), ((), ())),
            preferred_element_type=jnp.float32)

        @pl.when(i == pl.num_programs(0) - 1)
        def _():
            gr = sums[...] / jnp.maximum(counts[...], 1.0)
            logits = jnp.dot(gr, wp_ref[0:d, :], preferred_element_type=jnp.float32)
            logits += jnp.dot(ep_ref[...], wp_ref[d:d + de, :],
                              preferred_element_type=jnp.float32)
            o_ref[...] = logits + bp_ref[...]

    return pl.pallas_call(
        body,
        grid=(n // bn,),
        in_specs=[
            pl.BlockSpec((bn, d), lambda i: (i, 0)),
            pl.BlockSpec((bn, d), lambda i: (i, 0)),
            pl.BlockSpec((bn, d), lambda i: (i, 0)),
            pl.BlockSpec((bn, _NC), lambda i: (i, 0)),
            pl.BlockSpec((bn, 1), lambda i: (i, 0)),
            pl.BlockSpec((g, de), lambda i: (0, 0)),
            pl.BlockSpec((1, d), lambda i: (0, 0)),
            pl.BlockSpec((d + de, c), lambda i: (0, 0)),
            pl.BlockSpec((1, c), lambda i: (0, 0)),
        ],
        out_specs=pl.BlockSpec((g, c), lambda i: (0, 0)),
        out_shape=jax.ShapeDtypeStruct((g, c), jnp.float32),
        scratch_shapes=[
            pltpu.VMEM((g, d), jnp.float32),
            pltpu.VMEM((g, 1), jnp.float32),
        ],
    )(s0, s1, hs, deg_t, batch2, edge_pool, b2, wp, bp2)


def kernel(x, edge_index, weights, batch, edge_pool, W, b, Wp, bp):
    n, d = x.shape
    e = weights.shape[0]
    nw = _NC * _NS
    jpt = -(-e // (nw * _CH))
    jpt = -(-jpt // 8) * 8
    epad = nw * _CH * jpt
    pad = epad - e

    src = edge_index[0].astype(jnp.int32)
    dst = edge_index[1].astype(jnp.int32)
    pad_ids = jnp.arange(pad, dtype=jnp.int32) % n
    src_t = jnp.concatenate([src, pad_ids]).reshape(nw, jpt, _CH)
    dst_t = jnp.concatenate([dst, pad_ids]).reshape(nw, jpt, _CH)
    w_t = jnp.pad(weights, (0, pad)).reshape(nw, jpt, _CH)

    deg0, deg1 = _sc_degree(dst_t, w_t, n)
    deg_t = jnp.stack([deg0, deg1], axis=1)
    hs = _tc_hs(x, W, deg_t)
    s0, s1 = _sc_scatter(src_t, dst_t, w_t, hs, n)
    return _tc_final(
        s0, s1, hs, deg_t,
        batch.astype(jnp.int32).reshape(n, 1),
        edge_pool, b.reshape(1, d), Wp, bp.reshape(1, -1),
    )
